# baseline (speedup 1.0000x reference)
import functools

import jax
import jax.numpy as jnp
import numpy as np
from jax import lax
from jax.experimental import pallas as pl
from jax.experimental.pallas import tpu as pltpu
from jax.experimental.pallas import tpu_sc as plsc

EPS = 1e-5
B = 16384
D = 32
NW = 32
BPW = B // NW
CHUNK = 128
NCHUNK = BPW // CHUNK
NG = BPW // 16

_MAGIC = np.int32(0x5F3759DF)
_MANT = np.int32(0x007FFFFF)
_ONE_F = np.int32(0x3F800000)
_LN2 = 0.6931471805599453
_SQRT2 = 1.4142135623730951


def _bits(x):
    return lax.bitcast_convert_type(x, jnp.int32)


def _f32(i):
    return lax.bitcast_convert_type(i, jnp.float32)


def _sqrt(x):
    y = _f32(_MAGIC - (_bits(x) >> 1))
    y = y * (1.5 - 0.5 * x * y * y)
    y = y * (1.5 - 0.5 * x * y * y)
    y = y * (1.5 - 0.5 * x * y * y)
    return x * y


def _log(x):
    i = _bits(x)
    e = (i >> 23) - 127
    m = _f32((i & _MANT) | _ONE_F)
    big = m > _SQRT2
    m = jnp.where(big, 0.5 * m, m)
    e = jnp.where(big, e + 1, e)
    t = (m - 1.0) / (m + 1.0)
    t2 = t * t
    p = 2.0 * t * (1.0 + t2 * (1.0 / 3.0 + t2 * (0.2 + t2 * (1.0 / 7.0 + t2 * (1.0 / 9.0)))))
    return e.astype(jnp.float32) * _LN2 + p


def _tanh_pos(x):
    return 1.0 - 2.0 / (jnp.exp(2.0 * x) + 1.0)


def _unit_scale(n):
    return jnp.where(n >= 1.0, (1.0 - EPS) / jnp.maximum(n, 1e-10), jnp.float32(1.0))


def _score_group(s_hh, s_tt, s_rr, s_hw2, s_tr, s_ht, s_hr, b0, b1):
    n_h0 = _sqrt(s_hh)
    sc_h = _unit_scale(n_h0)
    sc_t = _unit_scale(_sqrt(s_tt))
    sc_r = _unit_scale(_sqrt(s_rr))

    n1 = jnp.clip(sc_h * n_h0, 1e-10, 1.0 - 1e-7)
    artanh = 0.5 * _log((1.0 + n1) / (1.0 - n1))
    alpha = artanh / n1 * sc_h

    rt_hw2 = _sqrt(s_hw2)
    nm = jnp.maximum(alpha * rt_hw2, 1e-10)
    mu = alpha * _tanh_pos(nm) / nm
    zeta = mu * _unit_scale(mu * rt_hw2)
    head2 = zeta * zeta * s_hw2

    xy = sc_t * sc_r * s_tr
    x2 = sc_t * sc_t * s_tt
    y2 = sc_r * sc_r * s_rr
    den = jnp.maximum(1.0 + 2.0 * xy + x2 * y2, 1e-10)
    a = (1.0 + 2.0 * xy + y2) * sc_t / den
    b = (1.0 - x2) * sc_r / den
    u2 = a * a * s_tt + 2.0 * a * b * s_tr + b * b * s_rr
    sc_u = _unit_scale(_sqrt(u2))
    a2 = a * sc_u
    b2 = b * sc_u
    tail2 = sc_u * sc_u * u2

    d2 = head2 + tail2 - 2.0 * zeta * (a2 * s_ht + b2 * s_hr)
    axay = (1.0 - head2) * (1.0 - tail2)
    z1 = jnp.maximum(2.0 * d2 / jnp.maximum(axay, 1e-10), 1e-7)
    dist = _log(1.0 + z1 + _sqrt(z1 * (z1 + 2.0)))
    return -dist + b0 + b1


def _body(eh, rvh, wfh, bias0, bias1, hidx, ridx, tidx, out,
          idx_h, idx_t, idx_r, rows_h, rows_t, rows_r, rows_w,
          b0_v, b1_v, out_v, sem):
    wid = lax.axis_index("s") * 2 + lax.axis_index("c")
    base = wid * BPW

    for j in range(NCHUNK):
        pltpu.sync_copy(hidx.at[pl.ds(base + j * CHUNK, CHUNK)], idx_h.at[j])
        pltpu.sync_copy(tidx.at[pl.ds(base + j * CHUNK, CHUNK)], idx_t.at[j])
        pltpu.sync_copy(ridx.at[pl.ds(base + j * CHUNK, CHUNK)], idx_r.at[j])

    copies = []
    for j in range(NCHUNK):
        sl = pl.ds(j * CHUNK, CHUNK)
        copies.append(pltpu.make_async_copy(eh.at[idx_h.at[j]], rows_h.at[sl], sem))
        copies.append(pltpu.make_async_copy(eh.at[idx_t.at[j]], rows_t.at[sl], sem))
        copies.append(pltpu.make_async_copy(rvh.at[idx_r.at[j]], rows_r.at[sl], sem))
        copies.append(pltpu.make_async_copy(wfh.at[idx_r.at[j]], rows_w.at[sl], sem))
        copies.append(pltpu.make_async_copy(bias0.at[idx_h.at[j]], b0_v.at[sl], sem))
        copies.append(pltpu.make_async_copy(bias1.at[idx_t.at[j]], b1_v.at[sl], sem))
    for c in copies:
        c.start()
    for c in copies:
        c.wait()

    def group(g, carry):
        row = g * 16 + lax.iota(jnp.int32, 16)
        z = jnp.zeros((16,), jnp.float32)
        s_hh = z; s_tt = z; s_rr = z; s_hw2 = z; s_tr = z; s_ht = z; s_hr = z
        for dd in range(D):
            dim = jnp.full((16,), dd, jnp.int32)
            hd = plsc.load_gather(rows_h, [row, dim])
            td = plsc.load_gather(rows_t, [row, dim])
            rd = plsc.load_gather(rows_r, [row, dim])
            wd = plsc.load_gather(rows_w, [row, dim])
            hw = hd * wd
            s_hh += hd * hd
            s_tt += td * td
            s_rr += rd * rd
            s_hw2 += hw * hw
            s_tr += td * rd
            s_ht += hw * td
            s_hr += hw * rd
        b0 = b0_v[pl.ds(g * 16, 16)]
        b1 = b1_v[pl.ds(g * 16, 16)]
        score = _score_group(s_hh, s_tt, s_rr, s_hw2, s_tr, s_ht, s_hr, b0, b1)
        out_v[pl.ds(g * 16, 16)] = score
        return carry

    lax.fori_loop(0, NG, group, 0)
    pltpu.sync_copy(out_v, out.at[pl.ds(base, BPW)])


_sc_call = functools.partial(
    pl.kernel,
    out_type=jax.ShapeDtypeStruct((B,), jnp.float32),
    mesh=plsc.VectorSubcoreMesh(core_axis_name="c", subcore_axis_name="s"),
    compiler_params=pltpu.CompilerParams(
        needs_layout_passes=False, use_tc_tiling_on_sc=False),
    scratch_types=[
        pltpu.VMEM((NCHUNK, CHUNK), jnp.int32),
        pltpu.VMEM((NCHUNK, CHUNK), jnp.int32),
        pltpu.VMEM((NCHUNK, CHUNK), jnp.int32),
        pltpu.VMEM((BPW, D), jnp.float32),
        pltpu.VMEM((BPW, D), jnp.float32),
        pltpu.VMEM((BPW, D), jnp.float32),
        pltpu.VMEM((BPW, D), jnp.float32),
        pltpu.VMEM((BPW,), jnp.float32),
        pltpu.VMEM((BPW,), jnp.float32),
        pltpu.VMEM((BPW,), jnp.float32),
        pltpu.SemaphoreType.DMA,
    ],
)(_body)


def kernel(Eh, rvh, weight_for_head, bias0, bias1, head_idx, rel_idx, tail_idx):
    return _sc_call(
        Eh, rvh, weight_for_head,
        bias0, bias1,
        head_idx.astype(jnp.int32), rel_idx.astype(jnp.int32),
        tail_idx.astype(jnp.int32),
    )
